# Initial kernel scaffold; baseline (speedup 1.0000x reference)
import jax
import jax.numpy as jnp
from jax.experimental import pallas as pl


def kernel(x, W1, b1, g1, beta1, W2, b2, g2, beta2, fc1w, fc1b, fc2w, fc2b, edge_index, batch):
    raise NotImplementedError("write your pallas kernel here")


# speedup vs baseline: 36.1403x; 36.1403x over previous
import functools

import jax
import jax.numpy as jnp
import numpy as np
from jax import lax
from jax.experimental import pallas as pl
from jax.experimental.pallas import tpu as pltpu
from jax.experimental.pallas import tpu_sc as plsc

NN = 10000
EE = 320000
DIN = 128
HH = 64
GG = 64
CC = 2
BN_EPS_ = 1e-5

NPAD = 10240
RT = NPAD // 16
K = 80
CH = (EE // 32) // K
RB = 2000


def _sc_mesh():
    return plsc.VectorSubcoreMesh(core_axis_name="c", subcore_axis_name="s")


_SC_PARAMS = pltpu.CompilerParams(use_tc_tiling_on_sc=False)


def _deg_body(dst_hbm, out_hbm, didx_v, ones_v, zrow_v, deg_sp):
    c = lax.axis_index("c")
    s = lax.axis_index("s")
    wid = s * 2 + c
    for i in range(K // 16):
        ones_v[pl.ds(i * 16, 16)] = jnp.ones((16,), jnp.float32)
        zrow_v[pl.ds(i * 16, 16)] = jnp.zeros((16,), jnp.float32)
    for j in range(RT // K):
        pltpu.sync_copy(zrow_v, deg_sp.at[pl.ds(s * RT + j * K, K)])
    pltpu.sync_copy(dst_hbm.at[wid], didx_v)
    plsc.subcore_barrier()

    def body(i, carry):
        pltpu.sync_copy(ones_v, deg_sp.at[didx_v.at[i]], add=True)
        return carry

    lax.fori_loop(0, CH, body, 0)
    plsc.subcore_barrier()
    pltpu.sync_copy(deg_sp.at[pl.ds(s * RT, RT)],
                    out_hbm.at[c].at[pl.ds(s * RT, RT)])


@jax.jit
def _deg_call(dst3d):
    return pl.kernel(
        _deg_body,
        out_type=jax.ShapeDtypeStruct((2, NPAD), jnp.float32),
        mesh=_sc_mesh(),
        scratch_types=[
            pltpu.VMEM((CH, K), jnp.int32),
            pltpu.VMEM((K,), jnp.float32),
            pltpu.VMEM((K,), jnp.float32),
            pltpu.VMEM_SHARED((NPAD,), jnp.float32),
        ],
        compiler_params=_SC_PARAMS,
    )(dst3d)


def _agg_body(u_hbm, sidx_hbm, didx_hbm, out_hbm,
              sidx_v, didx_v, rows0, rows1, zbuf, agg_sp, sem0, sem1):
    c = lax.axis_index("c")
    s = lax.axis_index("s")
    wid = s * 2 + c
    for r in range(16):
        for k in range(HH // 16):
            zbuf[r, pl.ds(k * 16, 16)] = jnp.zeros((16,), jnp.float32)
    for j in range(RT // 16):
        pltpu.sync_copy(zbuf, agg_sp.at[pl.ds(s * RT + j * 16, 16)])
    pltpu.sync_copy(sidx_hbm.at[wid], sidx_v)
    pltpu.sync_copy(didx_hbm.at[wid], didx_v)
    plsc.subcore_barrier()

    pltpu.make_async_copy(u_hbm.at[sidx_v.at[0]], rows0, sem0).start()

    def body(k, carry):
        i0 = 2 * k
        i1 = 2 * k + 1
        pltpu.make_async_copy(u_hbm.at[sidx_v.at[i1]], rows1, sem1).start()
        pltpu.make_async_copy(u_hbm.at[sidx_v.at[i0]], rows0, sem0).wait()
        pltpu.sync_copy(rows0, agg_sp.at[didx_v.at[i0]], add=True)
        pltpu.make_async_copy(u_hbm.at[sidx_v.at[i1 + 1]], rows0, sem0).start()
        pltpu.make_async_copy(u_hbm.at[sidx_v.at[i1]], rows1, sem1).wait()
        pltpu.sync_copy(rows1, agg_sp.at[didx_v.at[i1]], add=True)
        return carry

    lax.fori_loop(0, CH // 2, body, 0)
    pltpu.make_async_copy(u_hbm.at[sidx_v.at[CH - 1]], rows0, sem0).wait()
    pltpu.sync_copy(rows0, agg_sp.at[didx_v.at[CH - 1]], add=True)
    plsc.subcore_barrier()
    pltpu.sync_copy(agg_sp.at[pl.ds(s * RT, RT)],
                    out_hbm.at[c].at[pl.ds(s * RT, RT)])


@jax.jit
def _agg_call(u, src3d, dst3d):
    return pl.kernel(
        _agg_body,
        out_type=jax.ShapeDtypeStruct((2, NPAD, HH), jnp.float32),
        mesh=_sc_mesh(),
        scratch_types=[
            pltpu.VMEM((CH, K), jnp.int32),
            pltpu.VMEM((CH, K), jnp.int32),
            pltpu.VMEM((K, HH), jnp.float32),
            pltpu.VMEM((K, HH), jnp.float32),
            pltpu.VMEM((16, HH), jnp.float32),
            pltpu.VMEM_SHARED((NPAD, HH), jnp.float32),
            pltpu.SemaphoreType.DMA,
            pltpu.SemaphoreType.DMA,
        ],
        compiler_params=_SC_PARAMS,
    )(u, src3d, dst3d)


def _mm1_body(x_ref, w_ref, deg_ref, u_ref, dinv_ref):
    dinv = lax.rsqrt(jnp.maximum(deg_ref[...] + 1.0, 1.0))
    h = jnp.dot(x_ref[...], w_ref[...], preferred_element_type=jnp.float32)
    dinv_b = jnp.broadcast_to(dinv, (RB, HH))
    u_ref[...] = h * dinv_b
    dinv_ref[...] = dinv_b


@jax.jit
def _mm1_call(x, W1, degsum):
    return pl.pallas_call(
        _mm1_body,
        grid=(NN // RB,),
        in_specs=[
            pl.BlockSpec((RB, DIN), lambda i: (i, 0)),
            pl.BlockSpec((DIN, HH), lambda i: (0, 0)),
            pl.BlockSpec((RB, 1), lambda i: (i, 0)),
        ],
        out_specs=[
            pl.BlockSpec((RB, HH), lambda i: (i, 0)),
            pl.BlockSpec((RB, HH), lambda i: (i, 0)),
        ],
        out_shape=[
            jax.ShapeDtypeStruct((NN, HH), jnp.float32),
            jax.ShapeDtypeStruct((NN, HH), jnp.float32),
        ],
    )(x, W1, degsum)


def _post1_body(agg_ref, u_ref, dinv_ref, g_ref, beta_ref, b_ref, w2_ref,
                h1_ref, u2_ref):
    sconst = np.float32(1.0 / np.sqrt(1.0 + BN_EPS_))
    sc = g_ref[...] * sconst
    agg = agg_ref[0] + agg_ref[1]
    y = dinv_ref[...] * (agg + u_ref[...])
    y = y * sc + (b_ref[...] * sc + beta_ref[...])
    h1 = jnp.maximum(y, 0.0)
    h1_ref[...] = h1
    h2 = jnp.dot(h1, w2_ref[...], preferred_element_type=jnp.float32)
    u2_ref[...] = dinv_ref[...] * h2


@jax.jit
def _post1_call(aggp, u1, dinv_bc, g1, beta1, b1, W2):
    return pl.pallas_call(
        _post1_body,
        grid=(NN // RB,),
        in_specs=[
            pl.BlockSpec((2, RB, HH), lambda i: (0, i, 0)),
            pl.BlockSpec((RB, HH), lambda i: (i, 0)),
            pl.BlockSpec((RB, HH), lambda i: (i, 0)),
            pl.BlockSpec((1, HH), lambda i: (0, 0)),
            pl.BlockSpec((1, HH), lambda i: (0, 0)),
            pl.BlockSpec((1, HH), lambda i: (0, 0)),
            pl.BlockSpec((HH, HH), lambda i: (0, 0)),
        ],
        out_specs=[
            pl.BlockSpec((RB, HH), lambda i: (i, 0)),
            pl.BlockSpec((RB, HH), lambda i: (i, 0)),
        ],
        out_shape=[
            jax.ShapeDtypeStruct((NN, HH), jnp.float32),
            jax.ShapeDtypeStruct((NN, HH), jnp.float32),
        ],
    )(aggp, u1, dinv_bc, g1, beta1, b1, W2)


def _final_body(agg_ref, u_ref, res_ref, dinv_ref, g_ref, beta_ref, b_ref,
                batch_ref, fc1w_ref, fc1b_ref, fc2w_ref, fc2b_ref,
                out_ref, acc, cnt):
    i = pl.program_id(0)

    @pl.when(i == 0)
    def _():
        acc[...] = jnp.zeros_like(acc)
        cnt[...] = jnp.zeros_like(cnt)

    sconst = np.float32(1.0 / np.sqrt(1.0 + BN_EPS_))
    sc = g_ref[...] * sconst
    agg = agg_ref[0] + agg_ref[1]
    y = dinv_ref[...] * (agg + u_ref[...])
    y = y * sc + (b_ref[...] * sc + beta_ref[...])
    h2 = jnp.maximum(y, 0.0) + res_ref[...]

    gids = lax.broadcasted_iota(jnp.int32, (1, GG), 1)
    pt = (batch_ref[...] == gids).astype(jnp.float32)
    acc[...] += lax.dot_general(pt, h2, (((0,), (0,)), ((), ())),
                                preferred_element_type=jnp.float32)
    ones_col = jnp.ones((RB, 1), jnp.float32)
    cnt[...] += lax.dot_general(pt, ones_col, (((0,), (0,)), ((), ())),
                                preferred_element_type=jnp.float32)

    @pl.when(i == pl.num_programs(0) - 1)
    def _():
        pooled = acc[...] / jnp.maximum(cnt[...], 1.0)
        z = jnp.dot(pooled, fc1w_ref[...], preferred_element_type=jnp.float32)
        z = jnp.maximum(z + fc1b_ref[...], 0.0)
        logits = jnp.dot(z, fc2w_ref[...], preferred_element_type=jnp.float32)
        logits = logits + fc2b_ref[...]
        m = jnp.max(logits, axis=1, keepdims=True)
        lse = jnp.log(jnp.sum(jnp.exp(logits - m), axis=1, keepdims=True)) + m
        out_ref[...] = logits - lse


@jax.jit
def _final_call(aggp, u2, h1res, dinv_bc, g2, beta2, b2, batch2d,
                fc1w, fc1b, fc2w, fc2b):
    return pl.pallas_call(
        _final_body,
        grid=(NN // RB,),
        in_specs=[
            pl.BlockSpec((2, RB, HH), lambda i: (0, i, 0)),
            pl.BlockSpec((RB, HH), lambda i: (i, 0)),
            pl.BlockSpec((RB, HH), lambda i: (i, 0)),
            pl.BlockSpec((RB, HH), lambda i: (i, 0)),
            pl.BlockSpec((1, HH), lambda i: (0, 0)),
            pl.BlockSpec((1, HH), lambda i: (0, 0)),
            pl.BlockSpec((1, HH), lambda i: (0, 0)),
            pl.BlockSpec((RB, 1), lambda i: (i, 0)),
            pl.BlockSpec((HH, HH), lambda i: (0, 0)),
            pl.BlockSpec((1, HH), lambda i: (0, 0)),
            pl.BlockSpec((HH, CC), lambda i: (0, 0)),
            pl.BlockSpec((1, CC), lambda i: (0, 0)),
        ],
        out_specs=pl.BlockSpec((GG, CC), lambda i: (0, 0)),
        out_shape=jax.ShapeDtypeStruct((GG, CC), jnp.float32),
        scratch_shapes=[
            pltpu.VMEM((GG, HH), jnp.float32),
            pltpu.VMEM((GG, 1), jnp.float32),
        ],
    )(aggp, u2, h1res, dinv_bc, g2, beta2, b2, batch2d,
      fc1w, fc1b, fc2w, fc2b)


def kernel(x, W1, b1, g1, beta1, W2, b2, g2, beta2, fc1w, fc1b, fc2w, fc2b,
           edge_index, batch):
    src3d = edge_index[0].reshape(32, CH, K)
    dst3d = edge_index[1].reshape(32, CH, K)

    degp = _deg_call(dst3d)
    degsum = (degp[0, :NN] + degp[1, :NN]).reshape(NN, 1)

    u1, dinv_bc = _mm1_call(x, W1, degsum)

    aggp1 = _agg_call(u1, src3d, dst3d)
    h1, u2 = _post1_call(aggp1, u1, dinv_bc,
                         g1.reshape(1, HH), beta1.reshape(1, HH),
                         b1.reshape(1, HH), W2)

    aggp2 = _agg_call(u2, src3d, dst3d)
    out = _final_call(aggp2, u2, h1, dinv_bc,
                      g2.reshape(1, HH), beta2.reshape(1, HH),
                      b2.reshape(1, HH), batch.reshape(NN, 1),
                      fc1w, fc1b.reshape(1, HH), fc2w, fc2b.reshape(1, CC))
    return out
